# baseline (speedup 1.0000x reference)
import math

import jax
import jax.numpy as jnp
from jax import lax
from jax.experimental import pallas as pl
from jax.experimental.pallas import tpu as pltpu
from jax.experimental.pallas import tpu_sc as plsc

D_MODEL_K = 64
SCALE_K = math.sqrt(float(D_MODEL_K))

NUM_CORES = 2
NUM_SUBCORES = 16
NUM_WORKERS = NUM_CORES * NUM_SUBCORES
LANES = 16

PAIR = 2048
BBLK = 128
NBUF = 4


def _prep_kernel(t1_ref, t2_ref, o_ref):
    o_ref[:, :D_MODEL_K] = jnp.swapaxes(t1_ref[...], 0, 1) * SCALE_K
    o_ref[:, D_MODEL_K:] = jnp.swapaxes(t2_ref[...], 0, 1) * SCALE_K


def _post_kernel(g_ref, x_ref, o_ref):
    xb = x_ref[...]
    for i in range(g_ref.shape[0]):
        gbt = jnp.swapaxes(g_ref[i], 0, 1)
        h = (xb[i:i + 1] & PAIR) != 0
        o_ref[i] = jnp.where(h, gbt[D_MODEL_K:], gbt[:D_MODEL_K])


def _gather_kernel(idx_hbm, table_hbm, inter_hbm, idx_v, idx2_v, g_v, gsem, wsem):
    seq = idx_hbm.shape[0]
    batch = idx_hbm.shape[1]
    wid = lax.axis_index("c") * NUM_SUBCORES + lax.axis_index("s")
    b0 = wid * BBLK

    def prep_and_fire(gb, s):
        pltpu.sync_copy(idx_hbm.at[s, pl.ds(b0, BBLK)], idx_v.at[gb])

        @pl.loop(0, BBLK, step=LANES)
        def _(k):
            v = idx_v.at[gb].at[pl.ds(k, LANES)][...]
            idx2_v.at[gb].at[pl.ds(k, LANES)][...] = ((v >> 12) << 11) | (
                v & (PAIR - 1)
            )
        pltpu.async_copy(
            table_hbm.at[idx2_v.at[gb]], g_v.at[gb], gsem.at[gb]
        )

    def wait_gather(gb):
        pltpu.make_async_copy(
            table_hbm.at[idx2_v.at[gb]], g_v.at[gb], gsem.at[gb]
        ).wait()

    def fire_write(gb, s):
        dst = inter_hbm.at[pl.ds(s * batch + b0, BBLK)]
        pltpu.async_copy(g_v.at[gb], dst, wsem.at[gb])

    def wait_write(gb, s):
        dst = inter_hbm.at[pl.ds(s * batch + b0, BBLK)]
        pltpu.make_async_copy(g_v.at[gb], dst, wsem.at[gb]).wait()

    for s in range(NBUF - 1):
        prep_and_fire(s, s)

    @pl.loop(0, seq // NBUF)
    def _(g):
        for b in range(NBUF):
            s = g * NBUF + b
            pb = (b + NBUF - 1) % NBUF
            wait_gather(b)
            fire_write(b, s)

            @pl.when(s + NBUF - 1 < seq)
            def _():
                @pl.when(s >= 1)
                def _():
                    wait_write(pb, s - 1)

                prep_and_fire(pb, s + NBUF - 1)

    for t in range(NBUF):
        wait_write((seq - NBUF + t) % NBUF, seq - NBUF + t)


def kernel(x, table):
    batch, seq = x.shape
    vocab, d = table.shape
    assert batch == NUM_WORKERS * BBLK and d == D_MODEL_K
    assert seq % NBUF == 0

    idx = x.T
    table_t = table.T

    groups = -(-vocab // (2 * PAIR))
    t2_rows = groups * PAIR

    prep = pl.pallas_call(
        _prep_kernel,
        grid=(groups,),
        in_specs=[
            pl.BlockSpec((d, PAIR), lambda g: (0, 2 * g)),
            pl.BlockSpec(
                (d, PAIR),
                lambda g: (0, jnp.minimum(2 * g + 1, (vocab - 1) // PAIR)),
            ),
        ],
        out_specs=pl.BlockSpec((PAIR, 2 * d), lambda g: (g, 0)),
        out_shape=jax.ShapeDtypeStruct((t2_rows, 2 * d), table.dtype),
    )
    t2 = prep(table_t, table_t)

    mesh = plsc.VectorSubcoreMesh(core_axis_name="c", subcore_axis_name="s")
    gather = pl.kernel(
        _gather_kernel,
        out_type=jax.ShapeDtypeStruct((seq * batch, 2 * d), table.dtype),
        mesh=mesh,
        compiler_params=pltpu.CompilerParams(needs_layout_passes=False),
        scratch_types=[
            pltpu.VMEM((NBUF, BBLK), jnp.int32),
            pltpu.VMEM((NBUF, BBLK), jnp.int32),
            pltpu.VMEM((NBUF, BBLK, 2 * d), jnp.float32),
            pltpu.SemaphoreType.DMA((NBUF,)),
            pltpu.SemaphoreType.DMA((NBUF,)),
        ],
    )
    inter = gather(idx, t2)

    sblk, bblk = 8, 1024
    post = pl.pallas_call(
        _post_kernel,
        grid=(seq // sblk, batch // bblk),
        in_specs=[
            pl.BlockSpec((sblk, bblk, 2 * d), lambda si, bi: (si, bi, 0)),
            pl.BlockSpec((sblk, bblk), lambda si, bi: (si, bi)),
        ],
        out_specs=pl.BlockSpec((sblk, d, bblk), lambda si, bi: (si, 0, bi)),
        out_shape=jax.ShapeDtypeStruct((seq, d, batch), table.dtype),
    )
    out = post(inter.reshape(seq, batch, 2 * d), idx)

    return out.transpose(2, 0, 1)


# speedup vs baseline: 1.0019x; 1.0019x over previous
import math

import jax
import jax.numpy as jnp
from jax import lax
from jax.experimental import pallas as pl
from jax.experimental.pallas import tpu as pltpu
from jax.experimental.pallas import tpu_sc as plsc

D_MODEL_K = 64
SCALE_K = math.sqrt(float(D_MODEL_K))

NUM_CORES = 2
NUM_SUBCORES = 16
NUM_WORKERS = NUM_CORES * NUM_SUBCORES
LANES = 16

PAIR = 2048
BBLK = 128
NBUF = 4


def _prep_kernel(t1_ref, t2_ref, o_ref):
    o_ref[:, :D_MODEL_K] = jnp.swapaxes(t1_ref[...], 0, 1) * SCALE_K
    o_ref[:, D_MODEL_K:] = jnp.swapaxes(t2_ref[...], 0, 1) * SCALE_K


def _post_kernel(g_ref, x_ref, o_ref):
    xb = x_ref[...]
    for i in range(g_ref.shape[0]):
        gbt = jnp.swapaxes(g_ref[i], 0, 1)
        h = (xb[i:i + 1] & PAIR) != 0
        o_ref[i] = jnp.where(h, gbt[D_MODEL_K:], gbt[:D_MODEL_K])


def _post_kernel_acc(g_ref, x_ref, acc_ref, o_ref):
    del acc_ref
    _post_kernel(g_ref, x_ref, o_ref)


def _gather_kernel(idx_hbm, table_hbm, inter_hbm, idx_v, idx2_v, g_v, gsem, wsem):
    seq = idx_hbm.shape[0]
    batch = idx_hbm.shape[1]
    wid = lax.axis_index("c") * NUM_SUBCORES + lax.axis_index("s")
    b0 = wid * BBLK

    def prep_and_fire(gb, s):
        pltpu.sync_copy(idx_hbm.at[s, pl.ds(b0, BBLK)], idx_v.at[gb])

        @pl.loop(0, BBLK, step=LANES)
        def _(k):
            v = idx_v.at[gb].at[pl.ds(k, LANES)][...]
            idx2_v.at[gb].at[pl.ds(k, LANES)][...] = ((v >> 12) << 11) | (
                v & (PAIR - 1)
            )
        pltpu.async_copy(
            table_hbm.at[idx2_v.at[gb]], g_v.at[gb], gsem.at[gb]
        )

    def wait_gather(gb):
        pltpu.make_async_copy(
            table_hbm.at[idx2_v.at[gb]], g_v.at[gb], gsem.at[gb]
        ).wait()

    def fire_write(gb, s):
        dst = inter_hbm.at[pl.ds(s * batch + b0, BBLK)]
        pltpu.async_copy(g_v.at[gb], dst, wsem.at[gb])

    def wait_write(gb, s):
        dst = inter_hbm.at[pl.ds(s * batch + b0, BBLK)]
        pltpu.make_async_copy(g_v.at[gb], dst, wsem.at[gb]).wait()

    for s in range(NBUF - 1):
        prep_and_fire(s, s)

    @pl.loop(0, seq // NBUF)
    def _(g):
        for b in range(NBUF):
            s = g * NBUF + b
            pb = (b + NBUF - 1) % NBUF
            wait_gather(b)
            fire_write(b, s)

            @pl.when(s + NBUF - 1 < seq)
            def _():
                @pl.when(s >= 1)
                def _():
                    wait_write(pb, s - 1)

                prep_and_fire(pb, s + NBUF - 1)

    for t in range(NBUF):
        wait_write((seq - NBUF + t) % NBUF, seq - NBUF + t)


def kernel(x, table):
    batch, seq = x.shape
    vocab, d = table.shape
    assert batch == NUM_WORKERS * BBLK and d == D_MODEL_K
    assert seq % NBUF == 0

    idx = x.T
    table_t = table.T

    groups = -(-vocab // (2 * PAIR))
    t2_rows = groups * PAIR

    prep = pl.pallas_call(
        _prep_kernel,
        grid=(groups,),
        in_specs=[
            pl.BlockSpec((d, PAIR), lambda g: (0, 2 * g)),
            pl.BlockSpec(
                (d, PAIR),
                lambda g: (0, jnp.minimum(2 * g + 1, (vocab - 1) // PAIR)),
            ),
        ],
        out_specs=pl.BlockSpec((PAIR, 2 * d), lambda g: (g, 0)),
        out_shape=jax.ShapeDtypeStruct((t2_rows, 2 * d), table.dtype),
    )
    t2 = prep(table_t, table_t)

    mesh = plsc.VectorSubcoreMesh(core_axis_name="c", subcore_axis_name="s")

    def make_gather(seq_p):
        return pl.kernel(
            _gather_kernel,
            out_type=jax.ShapeDtypeStruct((seq_p * batch, 2 * d), table.dtype),
            mesh=mesh,
            compiler_params=pltpu.CompilerParams(needs_layout_passes=False),
            scratch_types=[
                pltpu.VMEM((NBUF, BBLK), jnp.int32),
                pltpu.VMEM((NBUF, BBLK), jnp.int32),
                pltpu.VMEM((NBUF, BBLK, 2 * d), jnp.float32),
                pltpu.SemaphoreType.DMA((NBUF,)),
                pltpu.SemaphoreType.DMA((NBUF,)),
            ],
        )

    sblk, bblk = 8, 1024
    part_sizes = (56, 48, 48, 48) if seq == 200 else (seq,)

    out = None
    s0 = 0
    for seq_p in part_sizes:
        idx_p = lax.slice_in_dim(idx, s0, s0 + seq_p, axis=0)
        inter_p = make_gather(seq_p)(idx_p, t2)
        g3 = inter_p.reshape(seq_p, batch, 2 * d)
        base = s0 // sblk
        if out is None:
            out = pl.pallas_call(
                _post_kernel,
                grid=(seq_p // sblk, batch // bblk),
                in_specs=[
                    pl.BlockSpec((sblk, bblk, 2 * d), lambda si, bi: (si, bi, 0)),
                    pl.BlockSpec((sblk, bblk), lambda si, bi: (si, bi)),
                ],
                out_specs=pl.BlockSpec(
                    (sblk, d, bblk), lambda si, bi, b=base: (b + si, 0, bi)
                ),
                out_shape=jax.ShapeDtypeStruct((seq, d, batch), table.dtype),
            )(g3, idx_p)
        else:
            out = pl.pallas_call(
                _post_kernel_acc,
                grid=(seq_p // sblk, batch // bblk),
                in_specs=[
                    pl.BlockSpec((sblk, bblk, 2 * d), lambda si, bi: (si, bi, 0)),
                    pl.BlockSpec((sblk, bblk), lambda si, bi: (si, bi)),
                    pl.BlockSpec(memory_space=pl.ANY),
                ],
                out_specs=pl.BlockSpec(
                    (sblk, d, bblk), lambda si, bi, b=base: (b + si, 0, bi)
                ),
                out_shape=jax.ShapeDtypeStruct((seq, d, batch), table.dtype),
                input_output_aliases={2: 0},
            )(g3, idx_p, out)
        s0 += seq_p

    return out.transpose(2, 0, 1)
